# Initial kernel scaffold; baseline (speedup 1.0000x reference)
import jax
import jax.numpy as jnp
from jax.experimental import pallas as pl


def kernel(x, Wf1, Wf2, bf, af, Wt1, Wt2, bt, at, W_fuse, b_fuse, W_ih, W_hh, b_ih, b_hh, W_head, b_head):
    raise NotImplementedError("write your pallas kernel here")


# speedup vs baseline: 3.3776x; 3.3776x over previous
import jax
import jax.numpy as jnp
from jax.experimental import pallas as pl
from jax.experimental.pallas import tpu as pltpu

B, W, F, H = 16, 150, 57, 2
HID = 150
BAND_K = 10
ALPHA = 0.2


def _leaky(u):
    return jnp.where(u >= 0, u, jnp.float32(ALPHA) * u)


def _mega_body(xf_ref, xw_ref,
               Wf1_ref, Wf2_ref, bf_ref, af_ref,
               Wt1_ref, Wt2_ref, bt_ref, at_ref,
               Wfu_f_ref, Wfu_t_ref, bfu_ref,
               WirT_ref, WizT_ref, WinT_ref,
               WhrT_ref, WhzT_ref, WhnT_ref,
               br_ref, bz_ref, bin_ref, bhn_ref,
               Whead_ref, bhead_ref,
               out_ref,
               gir_ref, giz_ref, gin_ref):
    f32 = jnp.float32
    xf = xf_ref[:]
    xw = xw_ref[:]

    Li = []
    Lj = []
    for h in range(H):
        Li.append(jnp.dot(xf, Wf1_ref[h], preferred_element_type=f32))
        Lj.append(jnp.dot(xf, Wf2_ref[h], preferred_element_type=f32)
                  + bf_ref[h:h + 1, :])
    af = [af_ref[h:h + 1, :].reshape(1, 1, W) for h in range(H)]

    feat_parts = []
    for b in range(B):
        r0, r1 = b * F, (b + 1) * F
        attn_sum = None
        for h in range(H):
            u = Li[h][r0:r1][:, None, :] + Lj[h][r0:r1][None, :, :]
            e = jnp.sum(_leaky(u) * af[h], axis=-1)
            e = e - jnp.max(e, axis=-1, keepdims=True)
            p = jnp.exp(e)
            attn = p / jnp.sum(p, axis=-1, keepdims=True)
            attn_sum = attn if attn_sum is None else attn_sum + attn
        hb = jnp.dot(jnp.float32(0.5) * attn_sum, xf[r0:r1],
                     preferred_element_type=f32)
        feat_parts.append(jax.nn.sigmoid(hb).T)
    h_featT = jnp.stack(feat_parts, axis=1).reshape(W * B, F)

    Ti = []
    Tj = []
    for h in range(H):
        Ti.append(jnp.dot(xw, Wt1_ref[h], preferred_element_type=f32))
        Tj.append(jnp.dot(xw, Wt2_ref[h], preferred_element_type=f32)
                  + bt_ref[h:h + 1, :])
    at = [at_ref[h:h + 1, :] for h in range(H)]

    tv = jax.lax.broadcasted_iota(jnp.int32, (W, B, 1), 0).reshape(W * B, 1)

    def shift_rows(m, o):
        s = o * B
        if s == 0:
            return m
        z = jnp.zeros((abs(s), m.shape[1]), f32)
        if s > 0:
            return jnp.concatenate([m[s:], z], axis=0)
        return jnp.concatenate([z, m[:s]], axis=0)

    offs = list(range(-BAND_K, BAND_K + 1))
    attn_avg = None
    e_cols = {h: [] for h in range(H)}
    for o in offs:
        valid = jnp.logical_and(tv + o >= 0, tv + o < W)
        for h in range(H):
            u = Ti[h] + shift_rows(Tj[h], o)
            ek = jnp.sum(_leaky(u) * at[h], axis=-1, keepdims=True)
            e_cols[h].append(jnp.where(valid, ek, jnp.float32(-1e9)))
    for h in range(H):
        e = jnp.concatenate(e_cols[h], axis=1)
        e = e - jnp.max(e, axis=-1, keepdims=True)
        p = jnp.exp(e)
        attn = p / jnp.sum(p, axis=-1, keepdims=True)
        attn_avg = attn if attn_avg is None else attn_avg + attn
    attn_avg = jnp.float32(0.5) * attn_avg

    acc = jnp.zeros((W * B, F), f32)
    for k, o in enumerate(offs):
        acc = acc + attn_avg[:, k:k + 1] * shift_rows(xw, o)
    h_time = jax.nn.sigmoid(acc)

    fused = (jnp.dot(h_featT, Wfu_f_ref[:], preferred_element_type=f32)
             + jnp.dot(h_time, Wfu_t_ref[:], preferred_element_type=f32)
             + bfu_ref[:])

    gir_ref[:] = jnp.dot(fused, WirT_ref[:], preferred_element_type=f32) + br_ref[:]
    giz_ref[:] = jnp.dot(fused, WizT_ref[:], preferred_element_type=f32) + bz_ref[:]
    gin_ref[:] = jnp.dot(fused, WinT_ref[:], preferred_element_type=f32) + bin_ref[:]

    WhrT = WhrT_ref[:]
    WhzT = WhzT_ref[:]
    WhnT = WhnT_ref[:]
    bhn = bhn_ref[:]

    def step(t, hprev):
        r0 = t * B
        gr = gir_ref[pl.ds(r0, B), :]
        gz = giz_ref[pl.ds(r0, B), :]
        gn = gin_ref[pl.ds(r0, B), :]
        hr = jnp.dot(hprev, WhrT, preferred_element_type=f32)
        hz = jnp.dot(hprev, WhzT, preferred_element_type=f32)
        hn = jnp.dot(hprev, WhnT, preferred_element_type=f32) + bhn
        r = jax.nn.sigmoid(gr + hr)
        z = jax.nn.sigmoid(gz + hz)
        n = jnp.tanh(gn + r * hn)
        return (1.0 - z) * n + z * hprev

    hT = jax.lax.fori_loop(0, W, step, jnp.zeros((B, HID), f32))

    out_ref[:] = (jnp.dot(hT, Whead_ref[:], preferred_element_type=f32)
                  + bhead_ref[:])


def kernel(x, Wf1, Wf2, bf, af, Wt1, Wt2, bt, at, W_fuse, b_fuse,
           W_ih, W_hh, b_ih, b_hh, W_head, b_head):
    f32 = jnp.float32
    xf = jnp.transpose(x, (0, 2, 1)).reshape(B * F, W)
    xw = jnp.transpose(x, (1, 0, 2)).reshape(W * B, F)

    W_ir, W_iz, W_in = W_ih[:HID], W_ih[HID:2 * HID], W_ih[2 * HID:]
    W_hr, W_hz, W_hn = W_hh[:HID], W_hh[HID:2 * HID], W_hh[2 * HID:]
    br = (b_ih[:HID] + b_hh[:HID]).reshape(1, HID)
    bz = (b_ih[HID:2 * HID] + b_hh[HID:2 * HID]).reshape(1, HID)
    bin_ = b_ih[2 * HID:].reshape(1, HID)
    bhn = b_hh[2 * HID:].reshape(1, HID)

    return pl.pallas_call(
        _mega_body,
        out_shape=jax.ShapeDtypeStruct((B, 3), f32),
        scratch_shapes=[pltpu.VMEM((W * B, HID), f32)] * 3,
    )(xf, xw,
      Wf1, Wf2, bf, af,
      Wt1, Wt2, bt, at,
      W_fuse[:F], W_fuse[F:], b_fuse.reshape(1, F),
      W_ir.T, W_iz.T, W_in.T,
      W_hr.T, W_hz.T, W_hn.T,
      br, bz, bin_, bhn,
      W_head, b_head.reshape(1, 3))


# speedup vs baseline: 3.4424x; 1.0192x over previous
import jax
import jax.numpy as jnp
from jax.experimental import pallas as pl
from jax.experimental.pallas import tpu as pltpu

B, W, F, H = 16, 150, 57, 2
HID = 150
BAND_K = 10
ALPHA = 0.2


def _leaky(u):
    return jnp.where(u >= 0, u, jnp.float32(ALPHA) * u)


def _mega_body(xf_ref, xw_ref,
               Wf1_ref, Wf2_ref, bf_ref, af_ref,
               Wt1_ref, Wt2_ref, bt_ref, at_ref,
               Wfu_f_ref, Wfu_t_ref, bfu_ref,
               WihC_ref, WhhC_ref, biC_ref, bhn_ref,
               Whead_ref, bhead_ref,
               out_ref,
               gic_ref):
    f32 = jnp.float32
    xf = xf_ref[:]
    xw = xw_ref[:]

    Li = []
    Lj = []
    for h in range(H):
        Li.append(jnp.dot(xf, Wf1_ref[h], preferred_element_type=f32))
        Lj.append(jnp.dot(xf, Wf2_ref[h], preferred_element_type=f32)
                  + bf_ref[h:h + 1, :])
    af = [af_ref[h:h + 1, :].reshape(1, 1, W) for h in range(H)]

    feat_parts = []
    for b in range(B):
        r0, r1 = b * F, (b + 1) * F
        attn_sum = None
        for h in range(H):
            u = Li[h][r0:r1][:, None, :] + Lj[h][r0:r1][None, :, :]
            e = jnp.sum(_leaky(u) * af[h], axis=-1)
            e = e - jnp.max(e, axis=-1, keepdims=True)
            p = jnp.exp(e)
            attn = p / jnp.sum(p, axis=-1, keepdims=True)
            attn_sum = attn if attn_sum is None else attn_sum + attn
        hb = jnp.dot(jnp.float32(0.5) * attn_sum, xf[r0:r1],
                     preferred_element_type=f32)
        feat_parts.append(jax.nn.sigmoid(hb).T)
    h_featT = jnp.stack(feat_parts, axis=1).reshape(W * B, F)

    Ti = []
    Tj = []
    for h in range(H):
        Ti.append(jnp.dot(xw, Wt1_ref[h], preferred_element_type=f32))
        Tj.append(jnp.dot(xw, Wt2_ref[h], preferred_element_type=f32)
                  + bt_ref[h:h + 1, :])
    at = [at_ref[h:h + 1, :] for h in range(H)]

    tv = jax.lax.broadcasted_iota(jnp.int32, (W, B, 1), 0).reshape(W * B, 1)

    def shift_rows(m, o):
        s = o * B
        if s == 0:
            return m
        z = jnp.zeros((abs(s), m.shape[1]), f32)
        if s > 0:
            return jnp.concatenate([m[s:], z], axis=0)
        return jnp.concatenate([z, m[:s]], axis=0)

    offs = list(range(-BAND_K, BAND_K + 1))
    attn_avg = None
    e_cols = {h: [] for h in range(H)}
    for o in offs:
        valid = jnp.logical_and(tv + o >= 0, tv + o < W)
        for h in range(H):
            u = Ti[h] + shift_rows(Tj[h], o)
            ek = jnp.sum(_leaky(u) * at[h], axis=-1, keepdims=True)
            e_cols[h].append(jnp.where(valid, ek, jnp.float32(-1e9)))
    for h in range(H):
        e = jnp.concatenate(e_cols[h], axis=1)
        e = e - jnp.max(e, axis=-1, keepdims=True)
        p = jnp.exp(e)
        attn = p / jnp.sum(p, axis=-1, keepdims=True)
        attn_avg = attn if attn_avg is None else attn_avg + attn
    attn_avg = jnp.float32(0.5) * attn_avg

    acc = jnp.zeros((W * B, F), f32)
    for k, o in enumerate(offs):
        acc = acc + attn_avg[:, k:k + 1] * shift_rows(xw, o)
    h_time = jax.nn.sigmoid(acc)

    fused = (jnp.dot(h_featT, Wfu_f_ref[:], preferred_element_type=f32)
             + jnp.dot(h_time, Wfu_t_ref[:], preferred_element_type=f32)
             + bfu_ref[:])

    gic_ref[:] = (jnp.dot(fused, WihC_ref[:], preferred_element_type=f32)
                  + biC_ref[:])

    WhhC = WhhC_ref[:]
    bhn = bhn_ref[:]

    def step(t, hprev):
        gi = gic_ref[pl.ds(t * B, B), :]
        gh = jnp.dot(hprev, WhhC, preferred_element_type=f32)
        r = jax.nn.sigmoid(gi[:, 0:HID] + gh[:, 0:HID])
        z = jax.nn.sigmoid(gi[:, 256:256 + HID] + gh[:, 256:256 + HID])
        hn = gh[:, 512:512 + HID] + bhn
        n = jnp.tanh(gi[:, 512:512 + HID] + r * hn)
        return (1.0 - z) * n + z * hprev

    hT = jax.lax.fori_loop(0, W, step, jnp.zeros((B, HID), f32),
                           unroll=5)

    out_ref[:] = (jnp.dot(hT, Whead_ref[:], preferred_element_type=f32)
                  + bhead_ref[:])


def kernel(x, Wf1, Wf2, bf, af, Wt1, Wt2, bt, at, W_fuse, b_fuse,
           W_ih, W_hh, b_ih, b_hh, W_head, b_head):
    f32 = jnp.float32
    xf = jnp.transpose(x, (0, 2, 1)).reshape(B * F, W)
    xw = jnp.transpose(x, (1, 0, 2)).reshape(W * B, F)

    def _slot(m):
        return jnp.pad(m, ((0, 0), (0, 256 - HID)))

    W_ir, W_iz, W_in = W_ih[:HID], W_ih[HID:2 * HID], W_ih[2 * HID:]
    W_hr, W_hz, W_hn = W_hh[:HID], W_hh[HID:2 * HID], W_hh[2 * HID:]
    WihC = jnp.concatenate([_slot(W_ir.T), _slot(W_iz.T), _slot(W_in.T)], 1)
    WhhC = jnp.concatenate([_slot(W_hr.T), _slot(W_hz.T), _slot(W_hn.T)], 1)
    br = (b_ih[:HID] + b_hh[:HID]).reshape(1, HID)
    bz = (b_ih[HID:2 * HID] + b_hh[HID:2 * HID]).reshape(1, HID)
    bin_ = b_ih[2 * HID:].reshape(1, HID)
    biC = jnp.concatenate([_slot(br), _slot(bz), _slot(bin_)], 1)
    bhn = b_hh[2 * HID:].reshape(1, HID)

    return pl.pallas_call(
        _mega_body,
        out_shape=jax.ShapeDtypeStruct((B, 3), f32),
        scratch_shapes=[pltpu.VMEM((W * B, 768), f32)],
    )(xf, xw,
      Wf1, Wf2, bf, af,
      Wt1, Wt2, bt, at,
      W_fuse[:F], W_fuse[F:], b_fuse.reshape(1, F),
      WihC, WhhC, biC, bhn,
      W_head, b_head.reshape(1, 3))
